# baseline (speedup 1.0000x reference)
import functools

import jax
import jax.numpy as jnp
from jax import lax
from jax.experimental import pallas as pl
from jax.experimental.pallas import tpu as pltpu
from jax.experimental.pallas import tpu_sc as plsc

VOCAB = 1000000
SEQ = 200
BATCH = 16384
EMB = 3
TOTAL = BATCH * SEQ

NC = 2
NS = 16
NW = NC * NS
PER_W = TOTAL // NW
CHUNK = 6400
NCHUNK = PER_W // CHUNK
KSUB = 8
SUB = CHUNK // KSUB


@functools.cache
def _make_gather():
  mesh = plsc.VectorSubcoreMesh(
      core_axis_name="c", subcore_axis_name="s", num_cores=NC, num_subcores=NS
  )
  @functools.partial(
      pl.kernel,
      mesh=mesh,
      out_type=jax.ShapeDtypeStruct((TOTAL, 8), jnp.float32),
      scratch_types=[
          pltpu.VMEM((CHUNK,), jnp.int32),
          pltpu.VMEM((CHUNK, 8), jnp.float32),
          pltpu.SemaphoreType.DMA,
      ],
      compiler_params=pltpu.CompilerParams(use_tc_tiling_on_sc=False),
  )
  def gather_kernel(x_hbm, emb_hbm, out_hbm, idx_v, rows_v, sem):
    wid = lax.axis_index("s") * NC + lax.axis_index("c")
    base = wid * PER_W

    def body(j, _):
      o = base + j * CHUNK
      pltpu.sync_copy(x_hbm.at[pl.ds(o, CHUNK)], idx_v)
      cps = []
      for i in range(KSUB):
        cps.append(
            pltpu.async_copy(
                emb_hbm.at[idx_v.at[pl.ds(i * SUB, SUB)]],
                rows_v.at[pl.ds(i * SUB, SUB), :],
                sem,
            )
        )
      for cp in cps:
        cp.wait()
      pltpu.sync_copy(rows_v, out_hbm.at[pl.ds(o, CHUNK), :])
      return 0

    lax.fori_loop(0, NCHUNK, body, 0)

  return gather_kernel


BB = 1024


def _mlp_body(g_ref, w1_ref, b1_ref, w2_ref, b2_ref, w3_ref, b3_ref, o_ref):
  h = jnp.dot(g_ref[...], w1_ref[...], preferred_element_type=jnp.float32)
  h = jnp.maximum(h + b1_ref[...], 0.0)
  h = jnp.dot(h, w2_ref[...], preferred_element_type=jnp.float32) + b2_ref[...]
  h = jnp.maximum(h, 0.0)
  z = jnp.dot(h, w3_ref[...], preferred_element_type=jnp.float32) + b3_ref[...]
  o_ref[...] = 1.0 / (1.0 + jnp.exp(-z))


def _mlp(g, w1p, b1, w2t, b2, w3t, b3):
  grid = BATCH // BB
  full = lambda shape: pl.BlockSpec(shape, lambda i: (0, 0))
  return pl.pallas_call(
      _mlp_body,
      grid=(grid,),
      in_specs=[
          pl.BlockSpec((BB, SEQ * 8), lambda i: (i, 0)),
          full((SEQ * 8, 10)),
          full((1, 10)),
          full((10, 5)),
          full((1, 5)),
          full((5, 3)),
          full((1, 3)),
      ],
      out_specs=pl.BlockSpec((BB, 3), lambda i: (i, 0)),
      out_shape=jax.ShapeDtypeStruct((BATCH, 3), jnp.float32),
  )(g, w1p, b1, w2t, b2, w3t, b3)


@jax.jit
def kernel(x, emb, W1, b1, W2, b2, W3, b3):
  x_flat = x.astype(jnp.int32).reshape(TOTAL)
  emb8 = jnp.pad(emb, ((0, 0), (0, 8 - EMB)))
  g8 = _make_gather()(x_flat, emb8)
  g = g8.reshape(BATCH, SEQ * 8)
  w1p = jnp.pad(W1.reshape(10, SEQ, EMB), ((0, 0), (0, 0), (0, 8 - EMB)))
  w1p = w1p.reshape(10, SEQ * 8).T
  return _mlp(
      g,
      w1p,
      b1.reshape(1, 10),
      W2.T,
      b2.reshape(1, 5),
      W3.T,
      b3.reshape(1, 3),
  )
